# baseline (speedup 1.0000x reference)
import functools

import jax
import jax.numpy as jnp
from jax import lax
from jax.experimental import pallas as pl
from jax.experimental.pallas import tpu as pltpu
from jax.experimental.pallas import tpu_sc as plsc

B = 1024
REPR_DIM = 4096
FEATURE_DIM = 512
HIDDEN_DIM = 1024
STATE_DIM = 64
GATE_DIM = 256
MOE_HIDDEN = 256
NUM_EXPERTS = 32
TOP_K = 4
ACTION_DIM = 12

BB = 256
NB = B // BB
M_TILE = 128
NBLK = 64
NSLOT = NBLK * M_TILE
NW = 32
TPW = B // NW


def _bdot(a, b):
    return jnp.dot(a.astype(jnp.bfloat16), b.astype(jnp.bfloat16),
                   preferred_element_type=jnp.float32)


def _k1_body(obs_ref, obs_sensor_ref, W_t_ref, b_t_ref, ln_g_ref, ln_b_ref,
             W_s1_ref, b_s1_ref, W_s2_ref, b_s2_ref, W_f1_ref, b_f1_ref,
             W_f2_ref, b_f2_ref, W_p1_ref, b_p1_ref, W_g1_ref, b_g1_ref,
             W_g2_ref, b_g2_ref,
             x_out_ref, topw_ref, slot2d_ref, be_ref, aux_out_ref,
             acc_ref, run_ref, topi_s_ref, rank_s_ref):
    b = pl.program_id(0)
    f32 = jnp.float32
    E = NUM_EXPERTS

    obs = obs_ref[...]
    h0 = _bdot(obs, W_t_ref[...]) + b_t_ref[...]
    mu = jnp.mean(h0, axis=-1, keepdims=True)
    var = jnp.mean((h0 - mu) ** 2, axis=-1, keepdims=True)
    h = jnp.tanh((h0 - mu) / jnp.sqrt(var + 1e-5) * ln_g_ref[...] + ln_b_ref[...])

    s1 = jax.nn.relu(_bdot(obs_sensor_ref[...], W_s1_ref[...]) + b_s1_ref[...])
    h = h + _bdot(s1, W_s2_ref[...]) + b_s2_ref[...]
    f1 = jax.nn.relu(_bdot(h, W_f1_ref[...]) + b_f1_ref[...])
    h = _bdot(f1, W_f2_ref[...]) + b_f2_ref[...]
    x = jax.nn.relu(_bdot(h, W_p1_ref[...]) + b_p1_ref[...])
    x_out_ref[...] = x

    g1 = jax.nn.relu(
        jnp.dot(x, W_g1_ref[...], preferred_element_type=f32) + b_g1_ref[...])
    gl = jnp.dot(g1, W_g2_ref[...], preferred_element_type=f32) + b_g2_ref[...]

    m = jnp.max(gl, axis=-1, keepdims=True)
    ex = jnp.exp(gl - m)
    p = ex / jnp.sum(ex, axis=-1, keepdims=True)

    @pl.when(b == 0)
    def _():
        acc_ref[...] = jnp.zeros_like(acc_ref)
        run_ref[...] = jnp.zeros_like(run_ref)

    tri = (lax.broadcasted_iota(jnp.int32, (E, E), 0)
           <= lax.broadcasted_iota(jnp.int32, (E, E), 1)).astype(f32)
    slt = (lax.broadcasted_iota(jnp.int32, (BB, BB), 0)
           > lax.broadcasted_iota(jnp.int32, (BB, BB), 1)).astype(f32)
    lane_e = lax.broadcasted_iota(jnp.int32, (1, E), 1).astype(f32)
    rem = p
    sel = jnp.zeros_like(p)
    idx_cols = []
    val_cols = []
    rank_cols = []
    for _ in range(TOP_K):
        mk = jnp.max(rem, axis=-1, keepdims=True)
        hit = (rem == mk).astype(f32)
        cs = jnp.dot(hit, tri, preferred_element_type=f32)
        first = hit * (cs == 1.0).astype(f32)
        idx_cols.append(jnp.sum(first * lane_e, axis=-1, keepdims=True))
        val_cols.append(jnp.sum(first * p, axis=-1, keepdims=True))
        wk = jnp.dot(slt, first, preferred_element_type=f32)
        run_row = run_ref[0:1, :]
        rank_cols.append(
            jnp.sum(first * (run_row + wk), axis=-1, keepdims=True))
        run_ref[0:1, :] = run_row + jnp.sum(first, axis=0, keepdims=True)
        sel = sel + first
        rem = rem - first * (rem + 1.0)
    topv = jnp.concatenate(val_cols, axis=-1)
    topsum = jnp.sum(topv, axis=-1, keepdims=True)
    topw_ref[...] = topv / topsum
    topi_s_ref[pl.ds(b * BB, BB), :] = jnp.concatenate(idx_cols, axis=-1)
    rank_s_ref[pl.ds(b * BB, BB), :] = jnp.concatenate(rank_cols, axis=-1)

    counts = jnp.sum(sel, axis=0, keepdims=True)
    psum = jnp.sum(p, axis=0, keepdims=True)
    part = jnp.concatenate([counts, psum, jnp.zeros((6, E), f32)], axis=0)
    acc_ref[...] += part
    aux = (E / (B * TOP_K * B)) * jnp.sum(acc_ref[0, :] * acc_ref[1, :])
    aux_out_ref[...] = jnp.full((8, 128), aux, f32)

    @pl.when(b == NB - 1)
    def _():
        g = acc_ref[0:1, :]
        pg = jnp.ceil(g / M_TILE) * M_TILE
        tri_strict = (lax.broadcasted_iota(jnp.int32, (E, E), 0)
                      < lax.broadcasted_iota(jnp.int32, (E, E), 1)).astype(f32)
        base = jnp.dot(pg, tri_strict, preferred_element_type=f32)

        lane_e_full = lax.broadcasted_iota(jnp.int32, (1, E), 1).astype(f32)
        slot_cols = []
        for k in range(TOP_K):
            oh = (topi_s_ref[:, k:k + 1] == lane_e_full).astype(f32)
            slot_col = (jnp.sum(oh * base, axis=-1, keepdims=True)
                        + rank_s_ref[:, k:k + 1])
            slot_cols.append(slot_col)
        slot2d_ref[...] = jnp.concatenate(slot_cols, axis=-1).astype(jnp.int32)

        startblk = base / M_TILE
        jcol = lax.broadcasted_iota(jnp.int32, (NBLK, 1), 0).astype(f32)
        cnt = jnp.sum((startblk <= jcol).astype(f32), axis=-1,
                      keepdims=True)
        be_ref[...] = jnp.broadcast_to((cnt - 1.0).reshape(1, NBLK),
                                       (8, NBLK)).astype(jnp.int32)


def _k3_body(be_ref, xs_ref, w1_ref, b1_ref, w2_ref, b2_ref, os_ref):
    eh = jax.nn.relu(_bdot(xs_ref[...], w1_ref[0]) + b1_ref[0])
    os_ref[...] = _bdot(eh, w2_ref[0]) + b2_ref[0]


def _k5_body(g_ref, topw_ref, W_p2_ref, b_p2_ref, std_ref,
             mu_ref, std_out_ref):
    f32 = jnp.float32
    tw = topw_ref[...]
    y = jnp.zeros((BB, HIDDEN_DIM), f32)
    for k in range(TOP_K):
        oh = (lax.broadcasted_iota(jnp.int32, (TOP_K, 1), 0) == k).astype(f32)
        wcol = jnp.dot(tw, oh, preferred_element_type=f32)
        y = y + wcol * g_ref[:, k * HIDDEN_DIM:(k + 1) * HIDDEN_DIM]
    muv = jnp.tanh(_bdot(jax.nn.relu(y), W_p2_ref[...]) + b_p2_ref[...])
    mu_ref[...] = muv
    std_out_ref[...] = jnp.full((BB, ACTION_DIM), std_ref[0, 0], f32)


def _sc_router_body(x_hbm, sk0_hbm, sk1_hbm, sk2_hbm, sk3_hbm, xs_hbm,
                    xrows_v, slotk_v, sem):
    w = lax.axis_index("s") * 2 + lax.axis_index("c")
    base = w * TPW
    pltpu.sync_copy(x_hbm.at[pl.ds(base, TPW)], xrows_v)
    for k, sk in enumerate((sk0_hbm, sk1_hbm, sk2_hbm, sk3_hbm)):
        pltpu.sync_copy(sk.at[pl.ds(base, TPW)], slotk_v.at[k])
    for k in range(TOP_K):
        pltpu.async_copy(xrows_v, xs_hbm.at[slotk_v.at[k]], sem).wait()


def _sc_gather_body(slots_hbm, os_hbm, g_hbm, slots_v, rows_v, sem):
    w = lax.axis_index("s") * 2 + lax.axis_index("c")
    pltpu.sync_copy(slots_hbm.at[pl.ds(w * 2, 2)], slots_v)
    for half in range(2):
        pltpu.async_copy(os_hbm.at[slots_v.at[half]], rows_v, sem).wait()
        pltpu.sync_copy(rows_v,
                        g_hbm.at[pl.ds(w * TPW * TOP_K + half * 64, 64)])


def kernel(obs, std, obs_sensor, W_t, b_t, ln_g, ln_b, W_s1, b_s1, W_s2, b_s2,
           W_f1, b_f1, W_f2, b_f2, W_p1, b_p1, W_g1, b_g1, W_g2, b_g2,
           W_e1, b_e1, W_e2, b_e2, W_p2, b_p2):
    f32 = jnp.float32
    i32 = jnp.int32
    bf16 = jnp.bfloat16
    r2 = lambda v: v.reshape(1, -1)
    const = lambda shape: pl.BlockSpec(shape, lambda b: tuple(0 for _ in shape))
    W_t, W_s1, W_s2, W_f1, W_f2, W_p1, W_e1, W_e2, W_p2 = (
        w.astype(bf16) for w in (W_t, W_s1, W_s2, W_f1, W_f2, W_p1,
                                 W_e1, W_e2, W_p2))

    x_out, topw, slot2d, be2d, auxmat = pl.pallas_call(
        _k1_body,
        grid=(NB,),
        in_specs=[
            pl.BlockSpec((BB, REPR_DIM), lambda b: (b, 0)),
            pl.BlockSpec((BB, STATE_DIM), lambda b: (b, 0)),
            const((REPR_DIM, FEATURE_DIM)),
            const((1, FEATURE_DIM)),
            const((1, FEATURE_DIM)),
            const((1, FEATURE_DIM)),
            const((STATE_DIM, HIDDEN_DIM)),
            const((1, HIDDEN_DIM)),
            const((HIDDEN_DIM, FEATURE_DIM)),
            const((1, FEATURE_DIM)),
            const((FEATURE_DIM, HIDDEN_DIM)),
            const((1, HIDDEN_DIM)),
            const((HIDDEN_DIM, FEATURE_DIM)),
            const((1, FEATURE_DIM)),
            const((FEATURE_DIM, HIDDEN_DIM)),
            const((1, HIDDEN_DIM)),
            const((HIDDEN_DIM, GATE_DIM)),
            const((1, GATE_DIM)),
            const((GATE_DIM, NUM_EXPERTS)),
            const((1, NUM_EXPERTS)),
        ],
        out_specs=[
            pl.BlockSpec((BB, HIDDEN_DIM), lambda b: (b, 0)),
            pl.BlockSpec((BB, TOP_K), lambda b: (b, 0)),
            const((B, TOP_K)),
            const((8, NBLK)),
            const((8, 128)),
        ],
        out_shape=[
            jax.ShapeDtypeStruct((B, HIDDEN_DIM), f32),
            jax.ShapeDtypeStruct((B, TOP_K), f32),
            jax.ShapeDtypeStruct((B, TOP_K), i32),
            jax.ShapeDtypeStruct((8, NBLK), i32),
            jax.ShapeDtypeStruct((8, 128), f32),
        ],
        scratch_shapes=[pltpu.VMEM((8, NUM_EXPERTS), f32),
                        pltpu.VMEM((1, NUM_EXPERTS), f32),
                        pltpu.VMEM((B, TOP_K), f32),
                        pltpu.VMEM((B, TOP_K), f32)],
    )(obs, obs_sensor, W_t, r2(b_t), r2(ln_g), r2(ln_b), W_s1, r2(b_s1),
      W_s2, r2(b_s2), W_f1, r2(b_f1), W_f2, r2(b_f2), W_p1, r2(b_p1),
      W_g1, r2(b_g1), W_g2, r2(b_g2))

    mesh = plsc.VectorSubcoreMesh(core_axis_name="c", subcore_axis_name="s")

    router = functools.partial(
        pl.kernel, mesh=mesh,
        out_type=jax.ShapeDtypeStruct((NSLOT, HIDDEN_DIM), f32),
        scratch_types=[
            pltpu.VMEM((TPW, HIDDEN_DIM), f32),
            pltpu.VMEM((TOP_K, TPW), i32),
            pltpu.SemaphoreType.DMA,
        ],
    )(_sc_router_body)
    x_sorted = router(x_out, slot2d[:, 0], slot2d[:, 1],
                      slot2d[:, 2], slot2d[:, 3])

    os_sorted = pl.pallas_call(
        _k3_body,
        grid_spec=pltpu.PrefetchScalarGridSpec(
            num_scalar_prefetch=1,
            grid=(NBLK,),
            in_specs=[
                pl.BlockSpec((M_TILE, HIDDEN_DIM), lambda j, be: (j, 0)),
                pl.BlockSpec((1, HIDDEN_DIM, MOE_HIDDEN),
                             lambda j, be: (be[j], 0, 0)),
                pl.BlockSpec((1, 1, MOE_HIDDEN), lambda j, be: (be[j], 0, 0)),
                pl.BlockSpec((1, MOE_HIDDEN, HIDDEN_DIM),
                             lambda j, be: (be[j], 0, 0)),
                pl.BlockSpec((1, 1, HIDDEN_DIM), lambda j, be: (be[j], 0, 0)),
            ],
            out_specs=pl.BlockSpec((M_TILE, HIDDEN_DIM), lambda j, be: (j, 0)),
        ),
        out_shape=jax.ShapeDtypeStruct((NSLOT, HIDDEN_DIM), f32),
    )(be2d[0], x_sorted, W_e1, b_e1[:, None, :], W_e2, b_e2[:, None, :])

    gatherer = functools.partial(
        pl.kernel, mesh=mesh,
        out_type=jax.ShapeDtypeStruct((B * TOP_K, HIDDEN_DIM), f32),
        scratch_types=[
            pltpu.VMEM((2, 64), i32),
            pltpu.VMEM((64, HIDDEN_DIM), f32),
            pltpu.SemaphoreType.DMA,
        ],
    )(_sc_gather_body)
    g = gatherer(slot2d.reshape(B * TOP_K // 64, 64), os_sorted)

    mu_out, std_out = pl.pallas_call(
        _k5_body,
        grid=(NB,),
        in_specs=[
            pl.BlockSpec((BB, TOP_K * HIDDEN_DIM), lambda b: (b, 0)),
            pl.BlockSpec((BB, TOP_K), lambda b: (b, 0)),
            const((HIDDEN_DIM, ACTION_DIM)),
            const((1, ACTION_DIM)),
            pl.BlockSpec(memory_space=pltpu.SMEM),
        ],
        out_specs=[
            pl.BlockSpec((BB, ACTION_DIM), lambda b: (b, 0)),
            pl.BlockSpec((BB, ACTION_DIM), lambda b: (b, 0)),
        ],
        out_shape=[
            jax.ShapeDtypeStruct((B, ACTION_DIM), f32),
            jax.ShapeDtypeStruct((B, ACTION_DIM), f32),
        ],
    )(g.reshape(B, TOP_K * HIDDEN_DIM), topw, W_p2, r2(b_p2),
      std.reshape(1, 1))

    return (mu_out, std_out, auxmat[0, 0])


# speedup vs baseline: 1.0309x; 1.0309x over previous
import functools

import jax
import jax.numpy as jnp
from jax import lax
from jax.experimental import pallas as pl
from jax.experimental.pallas import tpu as pltpu
from jax.experimental.pallas import tpu_sc as plsc

B = 1024
REPR_DIM = 4096
FEATURE_DIM = 512
HIDDEN_DIM = 1024
STATE_DIM = 64
GATE_DIM = 256
MOE_HIDDEN = 256
NUM_EXPERTS = 32
TOP_K = 4
ACTION_DIM = 12

BB = 256
NB = B // BB
M_TILE = 128
NBLK = 64
NSLOT = NBLK * M_TILE
NW = 32
TPW = B // NW


def _bdot(a, b):
    return jnp.dot(a.astype(jnp.bfloat16), b.astype(jnp.bfloat16),
                   preferred_element_type=jnp.float32)


def _k1_body(obs_ref, obs_sensor_ref, W_t_ref, b_t_ref, ln_g_ref, ln_b_ref,
             W_s1_ref, b_s1_ref, W_s2_ref, b_s2_ref, W_f1_ref, b_f1_ref,
             W_f2_ref, b_f2_ref, W_p1_ref, b_p1_ref, W_g1_ref, b_g1_ref,
             W_g2_ref, b_g2_ref,
             x_out_ref, topw_ref, slot2d_ref, be_ref, aux_out_ref,
             acc_ref, run_ref, topi_s_ref, rank_s_ref):
    b = pl.program_id(0)
    f32 = jnp.float32
    E = NUM_EXPERTS

    obs = obs_ref[...]
    h0 = _bdot(obs, W_t_ref[...]) + b_t_ref[...]
    mu = jnp.mean(h0, axis=-1, keepdims=True)
    var = jnp.mean((h0 - mu) ** 2, axis=-1, keepdims=True)
    h = jnp.tanh((h0 - mu) / jnp.sqrt(var + 1e-5) * ln_g_ref[...] + ln_b_ref[...])

    s1 = jax.nn.relu(_bdot(obs_sensor_ref[...], W_s1_ref[...]) + b_s1_ref[...])
    h = h + _bdot(s1, W_s2_ref[...]) + b_s2_ref[...]
    f1 = jax.nn.relu(_bdot(h, W_f1_ref[...]) + b_f1_ref[...])
    h = _bdot(f1, W_f2_ref[...]) + b_f2_ref[...]
    x = jax.nn.relu(_bdot(h, W_p1_ref[...]) + b_p1_ref[...])
    x_out_ref[...] = x

    g1 = jax.nn.relu(
        jnp.dot(x, W_g1_ref[...], preferred_element_type=f32) + b_g1_ref[...])
    gl = jnp.dot(g1, W_g2_ref[...], preferred_element_type=f32) + b_g2_ref[...]

    m = jnp.max(gl, axis=-1, keepdims=True)
    ex = jnp.exp(gl - m)
    p = ex / jnp.sum(ex, axis=-1, keepdims=True)

    @pl.when(b == 0)
    def _():
        acc_ref[...] = jnp.zeros_like(acc_ref)
        run_ref[...] = jnp.zeros_like(run_ref)

    tri = (lax.broadcasted_iota(jnp.int32, (E, E), 0)
           <= lax.broadcasted_iota(jnp.int32, (E, E), 1)).astype(f32)
    slt = (lax.broadcasted_iota(jnp.int32, (BB, BB), 0)
           > lax.broadcasted_iota(jnp.int32, (BB, BB), 1)).astype(f32)
    lane_e = lax.broadcasted_iota(jnp.int32, (1, E), 1).astype(f32)
    rem = p
    sel = jnp.zeros_like(p)
    idx_cols = []
    val_cols = []
    rank_cols = []
    for _ in range(TOP_K):
        mk = jnp.max(rem, axis=-1, keepdims=True)
        hit = (rem == mk).astype(f32)
        cs = jnp.dot(hit, tri, preferred_element_type=f32)
        first = hit * (cs == 1.0).astype(f32)
        idx_cols.append(jnp.sum(first * lane_e, axis=-1, keepdims=True))
        val_cols.append(jnp.sum(first * p, axis=-1, keepdims=True))
        wk = jnp.dot(slt, first, preferred_element_type=f32)
        run_row = run_ref[0:1, :]
        rank_cols.append(
            jnp.sum(first * (run_row + wk), axis=-1, keepdims=True))
        run_ref[0:1, :] = run_row + jnp.sum(first, axis=0, keepdims=True)
        sel = sel + first
        rem = rem - first * (rem + 1.0)
    topv = jnp.concatenate(val_cols, axis=-1)
    topsum = jnp.sum(topv, axis=-1, keepdims=True)
    topw_ref[...] = topv / topsum
    topi_s_ref[pl.ds(b * BB, BB), :] = jnp.concatenate(idx_cols, axis=-1)
    rank_s_ref[pl.ds(b * BB, BB), :] = jnp.concatenate(rank_cols, axis=-1)

    counts = jnp.sum(sel, axis=0, keepdims=True)
    psum = jnp.sum(p, axis=0, keepdims=True)
    part = jnp.concatenate([counts, psum, jnp.zeros((6, E), f32)], axis=0)
    acc_ref[...] += part
    aux = (E / (B * TOP_K * B)) * jnp.sum(acc_ref[0, :] * acc_ref[1, :])
    aux_out_ref[...] = jnp.full((8, 128), aux, f32)

    @pl.when(b == NB - 1)
    def _():
        g = acc_ref[0:1, :]
        pg = jnp.ceil(g / M_TILE) * M_TILE
        tri_strict = (lax.broadcasted_iota(jnp.int32, (E, E), 0)
                      < lax.broadcasted_iota(jnp.int32, (E, E), 1)).astype(f32)
        base = jnp.dot(pg, tri_strict, preferred_element_type=f32)

        lane_e_full = lax.broadcasted_iota(jnp.int32, (1, E), 1).astype(f32)
        slot_cols = []
        for k in range(TOP_K):
            oh = (topi_s_ref[:, k:k + 1] == lane_e_full).astype(f32)
            slot_col = (jnp.sum(oh * base, axis=-1, keepdims=True)
                        + rank_s_ref[:, k:k + 1])
            slot_cols.append(slot_col)
        slot2d_ref[...] = jnp.concatenate(slot_cols, axis=-1).astype(jnp.int32)

        startblk = base / M_TILE
        jcol = lax.broadcasted_iota(jnp.int32, (NBLK, 1), 0).astype(f32)
        cnt = jnp.sum((startblk <= jcol).astype(f32), axis=-1,
                      keepdims=True)
        bemap = jnp.broadcast_to((cnt - 1.0).reshape(1, NBLK), (8, NBLK))
        nblk = jnp.sum(pg) / M_TILE
        rowsel = (lax.broadcasted_iota(jnp.int32, (8, NBLK), 0) == 1)
        be_ref[...] = jnp.where(rowsel, nblk, bemap).astype(jnp.int32)


def _k3_body(be_ref, xs_ref, w1_ref, b1_ref, w2_ref, b2_ref, os_ref):
    j = pl.program_id(0)

    @pl.when(j < be_ref[NBLK])
    def _():
        eh = jax.nn.relu(_bdot(xs_ref[...], w1_ref[0]) + b1_ref[0])
        os_ref[...] = _bdot(eh, w2_ref[0]) + b2_ref[0]


def _k5_body(g_ref, topw_ref, W_p2_ref, b_p2_ref, std_ref,
             mu_ref, std_out_ref):
    f32 = jnp.float32
    tw = topw_ref[...]
    y = jnp.zeros((BB, HIDDEN_DIM), f32)
    for k in range(TOP_K):
        oh = (lax.broadcasted_iota(jnp.int32, (TOP_K, 1), 0) == k).astype(f32)
        wcol = jnp.dot(tw, oh, preferred_element_type=f32)
        y = y + wcol * g_ref[:, k * HIDDEN_DIM:(k + 1) * HIDDEN_DIM].astype(f32)
    muv = jnp.tanh(_bdot(jax.nn.relu(y), W_p2_ref[...]) + b_p2_ref[...])
    mu_ref[...] = muv
    std_out_ref[...] = jnp.full((BB, ACTION_DIM), std_ref[0, 0], f32)


def _sc_router_body(x_hbm, sk0_hbm, sk1_hbm, sk2_hbm, sk3_hbm, xs_hbm,
                    xrows_v, slotk_v, sem):
    w = lax.axis_index("s") * 2 + lax.axis_index("c")
    base = w * TPW
    loads = [pltpu.async_copy(x_hbm.at[pl.ds(base, TPW)], xrows_v, sem)]
    for k, sk in enumerate((sk0_hbm, sk1_hbm, sk2_hbm, sk3_hbm)):
        loads.append(pltpu.async_copy(sk.at[pl.ds(base, TPW)],
                                      slotk_v.at[k], sem))
    for c in loads:
        c.wait()
    scats = [pltpu.async_copy(xrows_v, xs_hbm.at[slotk_v.at[k]], sem)
             for k in range(TOP_K)]
    for c in scats:
        c.wait()


def _sc_gather_body(slots_hbm, os_hbm, g_hbm, slots_v, rows0_v, rows1_v,
                    sem_g, sem_p):
    w = lax.axis_index("s") * 2 + lax.axis_index("c")
    pltpu.sync_copy(slots_hbm.at[pl.ds(w * 4, 4)], slots_v)
    bufs = (rows0_v, rows1_v)
    CH = 32
    puts = [None, None]
    for c in range(4):
        buf = bufs[c % 2]
        if puts[c % 2] is not None:
            puts[c % 2].wait()
        get = pltpu.async_copy(os_hbm.at[slots_v.at[c]], buf, sem_g)
        get.wait()
        puts[c % 2] = pltpu.async_copy(
            buf, g_hbm.at[pl.ds(w * TPW * TOP_K + c * CH, CH)], sem_p)
    puts[0].wait()
    puts[1].wait()


def kernel(obs, std, obs_sensor, W_t, b_t, ln_g, ln_b, W_s1, b_s1, W_s2, b_s2,
           W_f1, b_f1, W_f2, b_f2, W_p1, b_p1, W_g1, b_g1, W_g2, b_g2,
           W_e1, b_e1, W_e2, b_e2, W_p2, b_p2):
    f32 = jnp.float32
    i32 = jnp.int32
    bf16 = jnp.bfloat16
    r2 = lambda v: v.reshape(1, -1)
    const = lambda shape: pl.BlockSpec(shape, lambda b: tuple(0 for _ in shape))
    W_t, W_s1, W_s2, W_f1, W_f2, W_p1, W_e1, W_e2, W_p2 = (
        w.astype(bf16) for w in (W_t, W_s1, W_s2, W_f1, W_f2, W_p1,
                                 W_e1, W_e2, W_p2))

    x_out, topw, slot2d, be2d, auxmat = pl.pallas_call(
        _k1_body,
        grid=(NB,),
        in_specs=[
            pl.BlockSpec((BB, REPR_DIM), lambda b: (b, 0)),
            pl.BlockSpec((BB, STATE_DIM), lambda b: (b, 0)),
            const((REPR_DIM, FEATURE_DIM)),
            const((1, FEATURE_DIM)),
            const((1, FEATURE_DIM)),
            const((1, FEATURE_DIM)),
            const((STATE_DIM, HIDDEN_DIM)),
            const((1, HIDDEN_DIM)),
            const((HIDDEN_DIM, FEATURE_DIM)),
            const((1, FEATURE_DIM)),
            const((FEATURE_DIM, HIDDEN_DIM)),
            const((1, HIDDEN_DIM)),
            const((HIDDEN_DIM, FEATURE_DIM)),
            const((1, FEATURE_DIM)),
            const((FEATURE_DIM, HIDDEN_DIM)),
            const((1, HIDDEN_DIM)),
            const((HIDDEN_DIM, GATE_DIM)),
            const((1, GATE_DIM)),
            const((GATE_DIM, NUM_EXPERTS)),
            const((1, NUM_EXPERTS)),
        ],
        out_specs=[
            pl.BlockSpec((BB, HIDDEN_DIM), lambda b: (b, 0)),
            pl.BlockSpec((BB, TOP_K), lambda b: (b, 0)),
            const((B, TOP_K)),
            const((8, NBLK)),
            const((8, 128)),
        ],
        out_shape=[
            jax.ShapeDtypeStruct((B, HIDDEN_DIM), f32),
            jax.ShapeDtypeStruct((B, TOP_K), f32),
            jax.ShapeDtypeStruct((B, TOP_K), i32),
            jax.ShapeDtypeStruct((8, NBLK), i32),
            jax.ShapeDtypeStruct((8, 128), f32),
        ],
        scratch_shapes=[pltpu.VMEM((8, NUM_EXPERTS), f32),
                        pltpu.VMEM((1, NUM_EXPERTS), f32),
                        pltpu.VMEM((B, TOP_K), f32),
                        pltpu.VMEM((B, TOP_K), f32)],
    )(obs, obs_sensor, W_t, r2(b_t), r2(ln_g), r2(ln_b), W_s1, r2(b_s1),
      W_s2, r2(b_s2), W_f1, r2(b_f1), W_f2, r2(b_f2), W_p1, r2(b_p1),
      W_g1, r2(b_g1), W_g2, r2(b_g2))

    mesh = plsc.VectorSubcoreMesh(core_axis_name="c", subcore_axis_name="s")

    router = functools.partial(
        pl.kernel, mesh=mesh,
        out_type=jax.ShapeDtypeStruct((NSLOT, HIDDEN_DIM), f32),
        scratch_types=[
            pltpu.VMEM((TPW, HIDDEN_DIM), f32),
            pltpu.VMEM((TOP_K, TPW), i32),
            pltpu.SemaphoreType.DMA,
        ],
    )(_sc_router_body)
    x_sorted = router(x_out, slot2d[:, 0], slot2d[:, 1],
                      slot2d[:, 2], slot2d[:, 3])

    def _clamp(j, be):
        return jnp.minimum(j, be[NBLK] - 1)

    os_sorted = pl.pallas_call(
        _k3_body,
        grid_spec=pltpu.PrefetchScalarGridSpec(
            num_scalar_prefetch=1,
            grid=(NBLK,),
            in_specs=[
                pl.BlockSpec((M_TILE, HIDDEN_DIM),
                             lambda j, be: (_clamp(j, be), 0)),
                pl.BlockSpec((1, HIDDEN_DIM, MOE_HIDDEN),
                             lambda j, be: (be[_clamp(j, be)], 0, 0)),
                pl.BlockSpec((1, 1, MOE_HIDDEN),
                             lambda j, be: (be[_clamp(j, be)], 0, 0)),
                pl.BlockSpec((1, MOE_HIDDEN, HIDDEN_DIM),
                             lambda j, be: (be[_clamp(j, be)], 0, 0)),
                pl.BlockSpec((1, 1, HIDDEN_DIM),
                             lambda j, be: (be[_clamp(j, be)], 0, 0)),
            ],
            out_specs=pl.BlockSpec((M_TILE, HIDDEN_DIM),
                                   lambda j, be: (_clamp(j, be), 0)),
        ),
        out_shape=jax.ShapeDtypeStruct((NSLOT, HIDDEN_DIM), f32),
    )(be2d[:2].reshape(-1), x_sorted, W_e1, b_e1[:, None, :],
      W_e2, b_e2[:, None, :])

    gatherer = functools.partial(
        pl.kernel, mesh=mesh,
        out_type=jax.ShapeDtypeStruct((B * TOP_K, HIDDEN_DIM), f32),
        scratch_types=[
            pltpu.VMEM((4, 32), i32),
            pltpu.VMEM((32, HIDDEN_DIM), f32),
            pltpu.VMEM((32, HIDDEN_DIM), f32),
            pltpu.SemaphoreType.DMA,
            pltpu.SemaphoreType.DMA,
        ],
    )(_sc_gather_body)
    g = gatherer(slot2d.reshape(B * TOP_K // 32, 32), os_sorted)

    mu_out, std_out = pl.pallas_call(
        _k5_body,
        grid=(NB,),
        in_specs=[
            pl.BlockSpec((BB, TOP_K * HIDDEN_DIM), lambda b: (b, 0)),
            pl.BlockSpec((BB, TOP_K), lambda b: (b, 0)),
            const((HIDDEN_DIM, ACTION_DIM)),
            const((1, ACTION_DIM)),
            pl.BlockSpec(memory_space=pltpu.SMEM),
        ],
        out_specs=[
            pl.BlockSpec((BB, ACTION_DIM), lambda b: (b, 0)),
            pl.BlockSpec((BB, ACTION_DIM), lambda b: (b, 0)),
        ],
        out_shape=[
            jax.ShapeDtypeStruct((B, ACTION_DIM), f32),
            jax.ShapeDtypeStruct((B, ACTION_DIM), f32),
        ],
    )(g.reshape(B, TOP_K * HIDDEN_DIM), topw, W_p2, r2(b_p2),
      std.reshape(1, 1))

    return (mu_out, std_out, auxmat[0, 0])


# speedup vs baseline: 1.3670x; 1.3260x over previous
import functools

import jax
import jax.numpy as jnp
from jax.experimental import pallas as pl
from jax.experimental.pallas import tpu as pltpu

B = 1024
REPR_DIM = 4096
FEATURE_DIM = 512
HIDDEN_DIM = 1024
STATE_DIM = 64
GATE_DIM = 256
MOE_HIDDEN = 256
NUM_EXPERTS = 32
TOP_K = 4
ACTION_DIM = 12

BB = 256
NB = B // BB


def _bdot(a, b):
    return jnp.dot(a.astype(jnp.bfloat16), b.astype(jnp.bfloat16),
                   preferred_element_type=jnp.float32)


def _k1_body(obs_ref, obs_sensor_ref, W_t_ref, b_t_ref, ln_g_ref, ln_b_ref,
             W_s1_ref, b_s1_ref, W_s2_ref, b_s2_ref, W_f1_ref, b_f1_ref,
             W_f2_ref, b_f2_ref, W_p1_ref, b_p1_ref, W_g1_ref, b_g1_ref,
             W_g2_ref, b_g2_ref,
             x_out_ref, combine_out_ref, aux_out_ref, acc_ref):
    b = pl.program_id(0)
    f32 = jnp.float32

    obs = obs_ref[...]
    h0 = _bdot(obs, W_t_ref[...]) + b_t_ref[...]
    mu = jnp.mean(h0, axis=-1, keepdims=True)
    var = jnp.mean((h0 - mu) ** 2, axis=-1, keepdims=True)
    h = jnp.tanh((h0 - mu) / jnp.sqrt(var + 1e-5) * ln_g_ref[...] + ln_b_ref[...])

    s1 = jax.nn.relu(_bdot(obs_sensor_ref[...], W_s1_ref[...]) + b_s1_ref[...])
    s = _bdot(s1, W_s2_ref[...]) + b_s2_ref[...]
    h = h + s

    f1 = jax.nn.relu(_bdot(h, W_f1_ref[...]) + b_f1_ref[...])
    h = _bdot(f1, W_f2_ref[...]) + b_f2_ref[...]

    x = jax.nn.relu(_bdot(h, W_p1_ref[...]) + b_p1_ref[...])
    x_out_ref[...] = x

    g1 = jax.nn.relu(
        jnp.dot(x, W_g1_ref[...], preferred_element_type=f32) + b_g1_ref[...])
    gl = jnp.dot(g1, W_g2_ref[...], preferred_element_type=f32) + b_g2_ref[...]

    m = jnp.max(gl, axis=-1, keepdims=True)
    ex = jnp.exp(gl - m)
    p = ex / jnp.sum(ex, axis=-1, keepdims=True)

    tri = (jax.lax.broadcasted_iota(jnp.int32, (NUM_EXPERTS, NUM_EXPERTS), 0)
           <= jax.lax.broadcasted_iota(jnp.int32, (NUM_EXPERTS, NUM_EXPERTS), 1)
           ).astype(f32)
    rem = p
    sel = jnp.zeros_like(p)
    for _ in range(TOP_K):
        mk = jnp.max(rem, axis=-1, keepdims=True)
        hit = (rem == mk).astype(f32)
        cs = jnp.dot(hit, tri, preferred_element_type=f32)
        first = hit * (cs == 1.0).astype(f32)
        sel = sel + first
        rem = rem - first * (rem + 1.0)
    topsum = jnp.sum(p * sel, axis=-1, keepdims=True)
    combine = p * sel / topsum
    combine_out_ref[...] = combine

    counts = jnp.sum(sel, axis=0, keepdims=True)
    psum = jnp.sum(p, axis=0, keepdims=True)
    part = jnp.concatenate([counts, psum, jnp.zeros((6, NUM_EXPERTS), f32)],
                           axis=0)

    @pl.when(b == 0)
    def _():
        acc_ref[...] = jnp.zeros_like(acc_ref)

    acc_ref[...] += part
    aux = (NUM_EXPERTS / (B * TOP_K * B)) * jnp.sum(
        acc_ref[0, :] * acc_ref[1, :])
    aux_out_ref[...] = jnp.full((8, 128), aux, f32)


def _k2_body(x_ref, combine_ref, W_e1_ref, b_e1_ref, W_e2_ref, b_e2_ref,
             W_p2_ref, b_p2_ref, std_ref, mu_ref, std_out_ref, y_ref):
    e = pl.program_id(0)
    f32 = jnp.float32

    @pl.when(e == 0)
    def _():
        y_ref[...] = jnp.zeros_like(y_ref)

    x = x_ref[...]
    eh = jax.nn.relu(_bdot(x, W_e1_ref[0]) + b_e1_ref[0])
    eo = _bdot(eh, W_e2_ref[0]) + b_e2_ref[0]
    e_onehot = (jax.lax.broadcasted_iota(jnp.int32, (NUM_EXPERTS, 1), 0)
                == e).astype(f32)
    c = jnp.dot(combine_ref[...], e_onehot, preferred_element_type=f32)
    y_ref[...] += c * eo

    @pl.when(e == NUM_EXPERTS - 1)
    def _():
        yw = jax.nn.relu(y_ref[...])
        muv = jnp.tanh(_bdot(yw, W_p2_ref[...]) + b_p2_ref[...])
        mu_ref[...] = muv
        std_out_ref[...] = jnp.full((B, ACTION_DIM), std_ref[0, 0], f32)


def kernel(obs, std, obs_sensor, W_t, b_t, ln_g, ln_b, W_s1, b_s1, W_s2, b_s2,
           W_f1, b_f1, W_f2, b_f2, W_p1, b_p1, W_g1, b_g1, W_g2, b_g2,
           W_e1, b_e1, W_e2, b_e2, W_p2, b_p2):
    f32 = jnp.float32
    bf16 = jnp.bfloat16
    r2 = lambda v: v.reshape(1, -1)
    W_t, W_s1, W_s2, W_f1, W_f2, W_p1, W_e1, W_e2, W_p2 = (
        w.astype(bf16) for w in (W_t, W_s1, W_s2, W_f1, W_f2, W_p1,
                                 W_e1, W_e2, W_p2))

    const = lambda shape: pl.BlockSpec(shape, lambda b: tuple(0 for _ in shape))
    x_out, combine, auxmat = pl.pallas_call(
        _k1_body,
        grid=(NB,),
        in_specs=[
            pl.BlockSpec((BB, REPR_DIM), lambda b: (b, 0)),
            pl.BlockSpec((BB, STATE_DIM), lambda b: (b, 0)),
            const((REPR_DIM, FEATURE_DIM)),
            const((1, FEATURE_DIM)),
            const((1, FEATURE_DIM)),
            const((1, FEATURE_DIM)),
            const((STATE_DIM, HIDDEN_DIM)),
            const((1, HIDDEN_DIM)),
            const((HIDDEN_DIM, FEATURE_DIM)),
            const((1, FEATURE_DIM)),
            const((FEATURE_DIM, HIDDEN_DIM)),
            const((1, HIDDEN_DIM)),
            const((HIDDEN_DIM, FEATURE_DIM)),
            const((1, FEATURE_DIM)),
            const((FEATURE_DIM, HIDDEN_DIM)),
            const((1, HIDDEN_DIM)),
            const((HIDDEN_DIM, GATE_DIM)),
            const((1, GATE_DIM)),
            const((GATE_DIM, NUM_EXPERTS)),
            const((1, NUM_EXPERTS)),
        ],
        out_specs=[
            pl.BlockSpec((BB, HIDDEN_DIM), lambda b: (b, 0)),
            pl.BlockSpec((BB, NUM_EXPERTS), lambda b: (b, 0)),
            pl.BlockSpec((8, 128), lambda b: (0, 0)),
        ],
        out_shape=[
            jax.ShapeDtypeStruct((B, HIDDEN_DIM), f32),
            jax.ShapeDtypeStruct((B, NUM_EXPERTS), f32),
            jax.ShapeDtypeStruct((8, 128), f32),
        ],
        scratch_shapes=[pltpu.VMEM((8, NUM_EXPERTS), f32)],
    )(obs, obs_sensor, W_t, r2(b_t), r2(ln_g), r2(ln_b), W_s1, r2(b_s1),
      W_s2, r2(b_s2), W_f1, r2(b_f1), W_f2, r2(b_f2), W_p1, r2(b_p1),
      W_g1, r2(b_g1), W_g2, r2(b_g2))

    mu_out, std_out = pl.pallas_call(
        _k2_body,
        grid=(NUM_EXPERTS,),
        in_specs=[
            pl.BlockSpec((B, HIDDEN_DIM), lambda e: (0, 0)),
            pl.BlockSpec((B, NUM_EXPERTS), lambda e: (0, 0)),
            pl.BlockSpec((1, HIDDEN_DIM, MOE_HIDDEN), lambda e: (e, 0, 0)),
            pl.BlockSpec((1, 1, MOE_HIDDEN), lambda e: (e, 0, 0)),
            pl.BlockSpec((1, MOE_HIDDEN, HIDDEN_DIM), lambda e: (e, 0, 0)),
            pl.BlockSpec((1, 1, HIDDEN_DIM), lambda e: (e, 0, 0)),
            pl.BlockSpec((HIDDEN_DIM, ACTION_DIM), lambda e: (0, 0)),
            pl.BlockSpec((1, ACTION_DIM), lambda e: (0, 0)),
            pl.BlockSpec(memory_space=pltpu.SMEM),
        ],
        out_specs=[
            pl.BlockSpec((B, ACTION_DIM), lambda e: (0, 0)),
            pl.BlockSpec((B, ACTION_DIM), lambda e: (0, 0)),
        ],
        out_shape=[
            jax.ShapeDtypeStruct((B, ACTION_DIM), f32),
            jax.ShapeDtypeStruct((B, ACTION_DIM), f32),
        ],
        scratch_shapes=[pltpu.VMEM((B, HIDDEN_DIM), f32)],
    )(x_out, combine, W_e1, b_e1[:, None, :], W_e2, b_e2[:, None, :],
      W_p2, r2(b_p2), std.reshape(1, 1))

    aux_loss = auxmat[0, 0]
    return (mu_out, std_out, aux_loss)


# speedup vs baseline: 1.8779x; 1.3738x over previous
import functools

import jax
import jax.numpy as jnp
from jax.experimental import pallas as pl
from jax.experimental.pallas import tpu as pltpu

B = 1024
REPR_DIM = 4096
FEATURE_DIM = 512
HIDDEN_DIM = 1024
STATE_DIM = 64
GATE_DIM = 256
MOE_HIDDEN = 256
NUM_EXPERTS = 32
TOP_K = 4
ACTION_DIM = 12

BB = 256
NB = B // BB


def _bdot(a, b):
    return jnp.dot(a.astype(jnp.bfloat16), b.astype(jnp.bfloat16),
                   preferred_element_type=jnp.float32)


def _k1_body(obs_ref, obs_sensor_ref, W_t_ref, b_t_ref, ln_g_ref, ln_b_ref,
             W_s1_ref, b_s1_ref, W_s2_ref, b_s2_ref, W_f1_ref, b_f1_ref,
             W_f2_ref, b_f2_ref, W_p1_ref, b_p1_ref, W_g1_ref, b_g1_ref,
             W_g2_ref, b_g2_ref,
             x_out_ref, combine_out_ref, aux_out_ref, acc_ref):
    b = pl.program_id(0)
    f32 = jnp.float32

    obs = obs_ref[...]
    h0 = _bdot(obs, W_t_ref[...]) + b_t_ref[...]
    mu = jnp.mean(h0, axis=-1, keepdims=True)
    var = jnp.mean((h0 - mu) ** 2, axis=-1, keepdims=True)
    h = jnp.tanh((h0 - mu) / jnp.sqrt(var + 1e-5) * ln_g_ref[...] + ln_b_ref[...])

    s1 = jax.nn.relu(_bdot(obs_sensor_ref[...], W_s1_ref[...]) + b_s1_ref[...])
    s = _bdot(s1, W_s2_ref[...]) + b_s2_ref[...]
    h = h + s

    f1 = jax.nn.relu(_bdot(h, W_f1_ref[...]) + b_f1_ref[...])
    h = _bdot(f1, W_f2_ref[...]) + b_f2_ref[...]

    x = jax.nn.relu(_bdot(h, W_p1_ref[...]) + b_p1_ref[...])
    x_out_ref[...] = x

    g1 = jax.nn.relu(
        jnp.dot(x, W_g1_ref[...], preferred_element_type=f32) + b_g1_ref[...])
    gl = jnp.dot(g1, W_g2_ref[...], preferred_element_type=f32) + b_g2_ref[...]

    m = jnp.max(gl, axis=-1, keepdims=True)
    ex = jnp.exp(gl - m)
    p = ex / jnp.sum(ex, axis=-1, keepdims=True)

    tri = (jax.lax.broadcasted_iota(jnp.int32, (NUM_EXPERTS, NUM_EXPERTS), 0)
           <= jax.lax.broadcasted_iota(jnp.int32, (NUM_EXPERTS, NUM_EXPERTS), 1)
           ).astype(f32)
    rem = p
    sel = jnp.zeros_like(p)
    for _ in range(TOP_K):
        mk = jnp.max(rem, axis=-1, keepdims=True)
        hit = (rem == mk).astype(f32)
        cs = jnp.dot(hit, tri, preferred_element_type=f32)
        first = hit * (cs == 1.0).astype(f32)
        sel = sel + first
        rem = rem - first * (rem + 1.0)
    topsum = jnp.sum(p * sel, axis=-1, keepdims=True)
    combine = p * sel / topsum
    combine_out_ref[...] = combine

    counts = jnp.sum(sel, axis=0, keepdims=True)
    psum = jnp.sum(p, axis=0, keepdims=True)
    part = jnp.concatenate([counts, psum, jnp.zeros((6, NUM_EXPERTS), f32)],
                           axis=0)

    @pl.when(b == 0)
    def _():
        acc_ref[...] = jnp.zeros_like(acc_ref)

    acc_ref[...] += part
    aux = (NUM_EXPERTS / (B * TOP_K * B)) * jnp.sum(
        acc_ref[0, :] * acc_ref[1, :])
    aux_out_ref[...] = jnp.full((8, 128), aux, f32)


def _k2_body(x_ref, combine_ref, W_e1_ref, b_e1_ref, W_e2_ref, b_e2_ref,
             W_p2_ref, b_p2_ref, std_ref, mu_ref, std_out_ref, y_ref):
    e = pl.program_id(0)
    f32 = jnp.float32

    @pl.when(e == 0)
    def _():
        y_ref[...] = jnp.zeros_like(y_ref)

    x = x_ref[...]
    eh = jax.nn.relu(_bdot(x, W_e1_ref[0]) + b_e1_ref[0])
    eo = _bdot(eh, W_e2_ref[0]) + b_e2_ref[0]
    e_onehot = (jax.lax.broadcasted_iota(jnp.int32, (NUM_EXPERTS, 1), 0)
                == e).astype(f32)
    c = jnp.dot(combine_ref[...], e_onehot, preferred_element_type=f32)
    y_ref[...] += c * eo

    @pl.when(e == NUM_EXPERTS - 1)
    def _():
        yw = jax.nn.relu(y_ref[...])
        muv = jnp.tanh(_bdot(yw, W_p2_ref[...]) + b_p2_ref[...])
        mu_ref[...] = muv
        std_out_ref[...] = jnp.full((B, ACTION_DIM), std_ref[0, 0], f32)


def kernel(obs, std, obs_sensor, W_t, b_t, ln_g, ln_b, W_s1, b_s1, W_s2, b_s2,
           W_f1, b_f1, W_f2, b_f2, W_p1, b_p1, W_g1, b_g1, W_g2, b_g2,
           W_e1, b_e1, W_e2, b_e2, W_p2, b_p2):
    f32 = jnp.float32
    r2 = lambda v: v.reshape(1, -1)

    const = lambda shape: pl.BlockSpec(shape, lambda b: tuple(0 for _ in shape))
    x_out, combine, auxmat = pl.pallas_call(
        _k1_body,
        grid=(NB,),
        in_specs=[
            pl.BlockSpec((BB, REPR_DIM), lambda b: (b, 0)),
            pl.BlockSpec((BB, STATE_DIM), lambda b: (b, 0)),
            const((REPR_DIM, FEATURE_DIM)),
            const((1, FEATURE_DIM)),
            const((1, FEATURE_DIM)),
            const((1, FEATURE_DIM)),
            const((STATE_DIM, HIDDEN_DIM)),
            const((1, HIDDEN_DIM)),
            const((HIDDEN_DIM, FEATURE_DIM)),
            const((1, FEATURE_DIM)),
            const((FEATURE_DIM, HIDDEN_DIM)),
            const((1, HIDDEN_DIM)),
            const((HIDDEN_DIM, FEATURE_DIM)),
            const((1, FEATURE_DIM)),
            const((FEATURE_DIM, HIDDEN_DIM)),
            const((1, HIDDEN_DIM)),
            const((HIDDEN_DIM, GATE_DIM)),
            const((1, GATE_DIM)),
            const((GATE_DIM, NUM_EXPERTS)),
            const((1, NUM_EXPERTS)),
        ],
        out_specs=[
            pl.BlockSpec((BB, HIDDEN_DIM), lambda b: (b, 0)),
            pl.BlockSpec((BB, NUM_EXPERTS), lambda b: (b, 0)),
            pl.BlockSpec((8, 128), lambda b: (0, 0)),
        ],
        out_shape=[
            jax.ShapeDtypeStruct((B, HIDDEN_DIM), f32),
            jax.ShapeDtypeStruct((B, NUM_EXPERTS), f32),
            jax.ShapeDtypeStruct((8, 128), f32),
        ],
        scratch_shapes=[pltpu.VMEM((8, NUM_EXPERTS), f32)],
    )(obs, obs_sensor, W_t, r2(b_t), r2(ln_g), r2(ln_b), W_s1, r2(b_s1),
      W_s2, r2(b_s2), W_f1, r2(b_f1), W_f2, r2(b_f2), W_p1, r2(b_p1),
      W_g1, r2(b_g1), W_g2, r2(b_g2))

    mu_out, std_out = pl.pallas_call(
        _k2_body,
        grid=(NUM_EXPERTS,),
        in_specs=[
            pl.BlockSpec((B, HIDDEN_DIM), lambda e: (0, 0)),
            pl.BlockSpec((B, NUM_EXPERTS), lambda e: (0, 0)),
            pl.BlockSpec((1, HIDDEN_DIM, MOE_HIDDEN), lambda e: (e, 0, 0)),
            pl.BlockSpec((1, 1, MOE_HIDDEN), lambda e: (e, 0, 0)),
            pl.BlockSpec((1, MOE_HIDDEN, HIDDEN_DIM), lambda e: (e, 0, 0)),
            pl.BlockSpec((1, 1, HIDDEN_DIM), lambda e: (e, 0, 0)),
            pl.BlockSpec((HIDDEN_DIM, ACTION_DIM), lambda e: (0, 0)),
            pl.BlockSpec((1, ACTION_DIM), lambda e: (0, 0)),
            pl.BlockSpec(memory_space=pltpu.SMEM),
        ],
        out_specs=[
            pl.BlockSpec((B, ACTION_DIM), lambda e: (0, 0)),
            pl.BlockSpec((B, ACTION_DIM), lambda e: (0, 0)),
        ],
        out_shape=[
            jax.ShapeDtypeStruct((B, ACTION_DIM), f32),
            jax.ShapeDtypeStruct((B, ACTION_DIM), f32),
        ],
        scratch_shapes=[pltpu.VMEM((B, HIDDEN_DIM), f32)],
    )(x_out, combine, W_e1, b_e1[:, None, :], W_e2, b_e2[:, None, :],
      W_p2, r2(b_p2), std.reshape(1, 1))

    aux_loss = auxmat[0, 0]
    return (mu_out, std_out, aux_loss)
